# baseline (speedup 1.0000x reference)
import jax
import jax.numpy as jnp
from jax.experimental import pallas as pl

NUM_HEADS = 8
OUT_FEATS = 64
ROW_BLOCK = 400


def _proj_kernel(x_ref, w_ref, o_ref):
    o_ref[:] = jnp.dot(x_ref[:], w_ref[:], preferred_element_type=jnp.float32)


def kernel(feat, edge_index, W_fc_self):
    del edge_index
    n, in_feats = feat.shape
    m = W_fc_self.shape[0]
    wt = W_fc_self.T
    out = pl.pallas_call(
        _proj_kernel,
        grid=(n // ROW_BLOCK,),
        in_specs=[
            pl.BlockSpec((ROW_BLOCK, in_feats), lambda i: (i, 0)),
            pl.BlockSpec((in_feats, m), lambda i: (0, 0)),
        ],
        out_specs=pl.BlockSpec((ROW_BLOCK, m), lambda i: (i, 0)),
        out_shape=jax.ShapeDtypeStruct((n, m), feat.dtype),
    )(feat, wt)
    return out.reshape(n, NUM_HEADS, OUT_FEATS)


# speedup vs baseline: 1.3115x; 1.3115x over previous
import jax
import jax.numpy as jnp
from jax.experimental import pallas as pl

NUM_HEADS = 8
OUT_FEATS = 64
ROW_BLOCK = 2000


def _proj_kernel(x_ref, w_ref, o_ref):
    o_ref[:] = jnp.dot(x_ref[:], w_ref[:], preferred_element_type=jnp.float32)


def kernel(feat, edge_index, W_fc_self):
    del edge_index
    n, in_feats = feat.shape
    m = W_fc_self.shape[0]
    wt = W_fc_self.T
    out = pl.pallas_call(
        _proj_kernel,
        grid=(n // ROW_BLOCK,),
        in_specs=[
            pl.BlockSpec((ROW_BLOCK, in_feats), lambda i: (i, 0)),
            pl.BlockSpec((in_feats, m), lambda i: (0, 0)),
        ],
        out_specs=pl.BlockSpec((ROW_BLOCK, m), lambda i: (i, 0)),
        out_shape=jax.ShapeDtypeStruct((n, m), feat.dtype),
    )(feat, wt)
    return out.reshape(n, NUM_HEADS, OUT_FEATS)
